# baseline (speedup 1.0000x reference)
import functools

import jax
import jax.numpy as jnp
from jax import lax
from jax.experimental import pallas as pl
from jax.experimental.pallas import tpu as pltpu
from jax.experimental.pallas import tpu_sc as plsc

B = 16384
EMB = 64
NC = 2
NS = 16
NW = NC * NS
BPW = B // NW


def _gather_body(uid_hbm, pid_hbm, nid_hbm, utab_hbm, mtab_hbm,
                 uout_hbm, pout_hbm, nout_hbm,
                 uidx_v, pidx_v, nidx_v, urow_v, prow_v, nrow_v, sem):
    wid = lax.axis_index("s") * NC + lax.axis_index("c")
    base = wid * BPW
    pltpu.sync_copy(uid_hbm.at[pl.ds(base, BPW)], uidx_v)
    pltpu.sync_copy(pid_hbm.at[pl.ds(base, BPW)], pidx_v)
    pltpu.sync_copy(nid_hbm.at[pl.ds(base, BPW)], nidx_v)
    cu = pltpu.async_copy(utab_hbm.at[uidx_v], urow_v, sem)
    cp = pltpu.async_copy(mtab_hbm.at[pidx_v], prow_v, sem)
    cn = pltpu.async_copy(mtab_hbm.at[nidx_v], nrow_v, sem)
    cu.wait()
    cp.wait()
    cn.wait()
    pltpu.sync_copy(urow_v, uout_hbm.at[pl.ds(base, BPW)])
    pltpu.sync_copy(prow_v, pout_hbm.at[pl.ds(base, BPW)])
    pltpu.sync_copy(nrow_v, nout_hbm.at[pl.ds(base, BPW)])


_sc_gather = functools.partial(
    pl.kernel,
    mesh=plsc.VectorSubcoreMesh(core_axis_name="c", subcore_axis_name="s"),
    out_type=[
        jax.ShapeDtypeStruct((B, EMB), jnp.float32),
        jax.ShapeDtypeStruct((B, EMB), jnp.float32),
        jax.ShapeDtypeStruct((B, EMB), jnp.float32),
    ],
    scratch_types=[
        pltpu.VMEM((BPW,), jnp.int32),
        pltpu.VMEM((BPW,), jnp.int32),
        pltpu.VMEM((BPW,), jnp.int32),
        pltpu.VMEM((BPW, EMB), jnp.float32),
        pltpu.VMEM((BPW, EMB), jnp.float32),
        pltpu.VMEM((BPW, EMB), jnp.float32),
        pltpu.SemaphoreType.DMA,
    ],
    compiler_params=pltpu.CompilerParams(use_tc_tiling_on_sc=False),
)(_gather_body)


BM = 2048


def _mlp_body(ue, pe, ne, uW1, ub1, uW2, ub2, mW1, mb1, mW2, mb2,
              uo, po, no):
    hu = jnp.maximum(
        jnp.dot(ue[...], uW1[...], preferred_element_type=jnp.float32) + ub1[...], 0.0)
    uo[...] = jnp.dot(hu, uW2[...], preferred_element_type=jnp.float32) + ub2[...]
    hp = jnp.maximum(
        jnp.dot(pe[...], mW1[...], preferred_element_type=jnp.float32) + mb1[...], 0.0)
    po[...] = jnp.dot(hp, mW2[...], preferred_element_type=jnp.float32) + mb2[...]
    hn = jnp.maximum(
        jnp.dot(ne[...], mW1[...], preferred_element_type=jnp.float32) + mb1[...], 0.0)
    no[...] = jnp.dot(hn, mW2[...], preferred_element_type=jnp.float32) + mb2[...]


def _mlp_towers(ue, pe, ne, uW1, ub1, uW2, ub2, mW1, mb1, mW2, mb2):
    emb_spec = pl.BlockSpec((BM, EMB), lambda i: (i, 0))
    w_spec = pl.BlockSpec((EMB, 64), lambda i: (0, 0))
    w2_spec = pl.BlockSpec((64, 32), lambda i: (0, 0))
    b1_spec = pl.BlockSpec((1, 64), lambda i: (0, 0))
    b2_spec = pl.BlockSpec((1, 32), lambda i: (0, 0))
    out_spec = pl.BlockSpec((BM, 32), lambda i: (i, 0))
    return pl.pallas_call(
        _mlp_body,
        grid=(B // BM,),
        in_specs=[emb_spec, emb_spec, emb_spec,
                  w_spec, b1_spec, w2_spec, b2_spec,
                  w_spec, b1_spec, w2_spec, b2_spec],
        out_specs=[out_spec, out_spec, out_spec],
        out_shape=[
            jax.ShapeDtypeStruct((B, 32), jnp.float32),
            jax.ShapeDtypeStruct((B, 32), jnp.float32),
            jax.ShapeDtypeStruct((B, 32), jnp.float32),
        ],
    )(ue, pe, ne, uW1, ub1.reshape(1, 64), uW2, ub2.reshape(1, 32),
      mW1, mb1.reshape(1, 64), mW2, mb2.reshape(1, 32))


def kernel(user_ids, pos_movie_ids, neg_movie_ids, user_table, movie_table,
           uW1, ub1, uW2, ub2, mW1, mb1, mW2, mb2):
    ue, pe, ne = _sc_gather(user_ids, pos_movie_ids, neg_movie_ids,
                            user_table, movie_table)
    return tuple(_mlp_towers(ue, pe, ne, uW1, ub1, uW2, ub2,
                             mW1, mb1, mW2, mb2))


# speedup vs baseline: 1.4892x; 1.4892x over previous
import functools

import jax
import jax.numpy as jnp
from jax import lax
from jax.experimental import pallas as pl
from jax.experimental.pallas import tpu as pltpu
from jax.experimental.pallas import tpu_sc as plsc

B = 16384
EMB = 64
NUM_ROWS = 1000000
NC = 2
NS = 16
NW = NC * NS
BPW = B // NW


CHUNK = 32


def _gather_body(uid_hbm, pid_hbm, nid_hbm, utab_hbm, mtab_hbm,
                 uout_hbm, pout_hbm, nout_hbm,
                 idx_v, out_v, sem):
    wid = lax.axis_index("s") * NC + lax.axis_index("c")
    base = wid * BPW
    ids = (uid_hbm, pid_hbm, nid_hbm)
    tabs = (utab_hbm, mtab_hbm, mtab_hbm)
    outs = (uout_hbm, pout_hbm, nout_hbm)
    for t in range(3):
        pltpu.sync_copy(ids[t].at[pl.ds(base, BPW)], idx_v)

        def chunk_body(c, _):
            cps = []
            for g in range(CHUNK // 16):
                x = idx_v[pl.ds(c * CHUNK + g * 16, 16)]
                for r in range(16):
                    row = x[r]
                    cps.append(pltpu.async_copy(
                        tabs[t].at[pl.ds(row, 1)],
                        out_v.at[pl.ds(g * 16 + r, 1)], sem))
            for cp in cps:
                cp.wait()
            pltpu.sync_copy(out_v,
                            outs[t].at[pl.ds(base + c * CHUNK, CHUNK)])
            return _

        lax.fori_loop(0, BPW // CHUNK, chunk_body, 0)


_sc_gather = functools.partial(
    pl.kernel,
    mesh=plsc.VectorSubcoreMesh(core_axis_name="c", subcore_axis_name="s"),
    out_type=[
        jax.ShapeDtypeStruct((B, EMB), jnp.float32),
        jax.ShapeDtypeStruct((B, EMB), jnp.float32),
        jax.ShapeDtypeStruct((B, EMB), jnp.float32),
    ],
    scratch_types=[
        pltpu.VMEM((BPW,), jnp.int32),
        pltpu.VMEM((CHUNK, EMB), jnp.float32),
        pltpu.SemaphoreType.DMA,
    ],
)(_gather_body)


BM = 2048


def _mlp_body(ue, pe, ne, uW1, ub1, uW2, ub2, mW1, mb1, mW2, mb2,
              uo, po, no):
    hu = jnp.maximum(
        jnp.dot(ue[...], uW1[...], preferred_element_type=jnp.float32) + ub1[...], 0.0)
    uo[...] = jnp.dot(hu, uW2[...], preferred_element_type=jnp.float32) + ub2[...]
    hp = jnp.maximum(
        jnp.dot(pe[...], mW1[...], preferred_element_type=jnp.float32) + mb1[...], 0.0)
    po[...] = jnp.dot(hp, mW2[...], preferred_element_type=jnp.float32) + mb2[...]
    hn = jnp.maximum(
        jnp.dot(ne[...], mW1[...], preferred_element_type=jnp.float32) + mb1[...], 0.0)
    no[...] = jnp.dot(hn, mW2[...], preferred_element_type=jnp.float32) + mb2[...]


def _mlp_towers(ue, pe, ne, uW1, ub1, uW2, ub2, mW1, mb1, mW2, mb2):
    emb_spec = pl.BlockSpec((BM, EMB), lambda i: (i, 0))
    w_spec = pl.BlockSpec((EMB, 64), lambda i: (0, 0))
    w2_spec = pl.BlockSpec((64, 32), lambda i: (0, 0))
    b1_spec = pl.BlockSpec((1, 64), lambda i: (0, 0))
    b2_spec = pl.BlockSpec((1, 32), lambda i: (0, 0))
    out_spec = pl.BlockSpec((BM, 32), lambda i: (i, 0))
    return pl.pallas_call(
        _mlp_body,
        grid=(B // BM,),
        in_specs=[emb_spec, emb_spec, emb_spec,
                  w_spec, b1_spec, w2_spec, b2_spec,
                  w_spec, b1_spec, w2_spec, b2_spec],
        out_specs=[out_spec, out_spec, out_spec],
        out_shape=[
            jax.ShapeDtypeStruct((B, 32), jnp.float32),
            jax.ShapeDtypeStruct((B, 32), jnp.float32),
            jax.ShapeDtypeStruct((B, 32), jnp.float32),
        ],
    )(ue, pe, ne, uW1, ub1.reshape(1, 64), uW2, ub2.reshape(1, 32),
      mW1, mb1.reshape(1, 64), mW2, mb2.reshape(1, 32))


def kernel(user_ids, pos_movie_ids, neg_movie_ids, user_table, movie_table,
           uW1, ub1, uW2, ub2, mW1, mb1, mW2, mb2):
    ue, pe, ne = _sc_gather(user_ids, pos_movie_ids, neg_movie_ids,
                            user_table, movie_table)
    return tuple(_mlp_towers(ue, pe, ne, uW1, ub1, uW2, ub2,
                             mW1, mb1, mW2, mb2))
